# baseline (speedup 1.0000x reference)
import functools
import math

import jax
import jax.numpy as jnp
from jax import lax
from jax.experimental import pallas as pl
from jax.experimental.pallas import tpu as pltpu
from jax.experimental.pallas import tpu_sc as plsc

N = 10000
D = 128
E = 320000
NUM_NEIGHBORS = 32.0
MIXING_ANGLE = math.pi / 8.0

NC = 2
NS = 16
NW = NC * NS
EPW = E // NW
K = 40
CHUNKS = EPW // K
PAIRS = CHUNKS // 2
RSPAN = 624
RLAST = N - 15 * RSPAN


def _tc_pre(x, w1):
    bm = 1000

    def body(x_ref, w1_ref, feat_ref, self_ref):
        t = jnp.dot(x_ref[...], w1_ref[...], preferred_element_type=jnp.float32)
        t = t * (1.0 / math.sqrt(D))
        feat_ref[...] = t[:, :D]
        self_ref[...] = t[:, D:]

    return pl.pallas_call(
        body,
        grid=(N // bm,),
        in_specs=[
            pl.BlockSpec((bm, D), lambda i: (i, 0)),
            pl.BlockSpec((D, 2 * D), lambda i: (0, 0)),
        ],
        out_specs=[
            pl.BlockSpec((bm, D), lambda i: (i, 0)),
            pl.BlockSpec((bm, D), lambda i: (i, 0)),
        ],
        out_shape=[
            jax.ShapeDtypeStruct((N, D), jnp.float32),
            jax.ShapeDtypeStruct((N, D), jnp.float32),
        ],
    )(x, w1)


def _sc_gather_scatter(feat, esrc, edst3, eattr3):
    mesh = plsc.VectorSubcoreMesh(core_axis_name="c", subcore_axis_name="s")

    @functools.partial(
        pl.kernel,
        mesh=mesh,
        out_type=jax.ShapeDtypeStruct((NC * N, D), jnp.float32),
        scratch_types=[
            pltpu.VMEM((K,), jnp.int32),
            pltpu.VMEM((K,), jnp.int32),
            pltpu.VMEM((K,), jnp.int32),
            pltpu.VMEM((K,), jnp.int32),
            pltpu.VMEM((K * 16,), jnp.float32),
            pltpu.VMEM((K * 16,), jnp.float32),
            pltpu.VMEM((K, D), jnp.float32),
            pltpu.VMEM((K, D), jnp.float32),
            pltpu.VMEM_SHARED((N, D), jnp.float32),
            pltpu.SemaphoreType.DMA,
            pltpu.SemaphoreType.DMA,
            pltpu.SemaphoreType.DMA,
            pltpu.SemaphoreType.DMA,
            pltpu.SemaphoreType.DMA,
            pltpu.SemaphoreType.DMA,
            pltpu.SemaphoreType.DMA,
            pltpu.SemaphoreType.DMA,
            pltpu.SemaphoreType.DMA,
            pltpu.SemaphoreType.DMA,
        ],
    )
    def k(feat_hbm, src_hbm, dst_hbm, attr_hbm, out_hbm,
          srcidx0, srcidx1, dstidx0, dstidx1, attr0, attr1,
          rows0, rows1, acc_sh,
          semi0, semi1, semd0, semd1, sema0, sema1,
          semg0, semg1, sems0, sems1):
        c = lax.axis_index("c")
        s = lax.axis_index("s")
        tid = c * NS + s
        ebase = tid * EPW

        roff = s * RSPAN

        def zero_rows(r, carry):
            for j in range(D // 16):
                rows0[r, pl.ds(j * 16, 16)] = jnp.zeros((16,), jnp.float32)
            return carry

        lax.fori_loop(0, K, zero_rows, 0)

        def fill_acc(i, carry):
            pltpu.async_copy(rows0, acc_sh.at[pl.ds(roff + i * K, K)], sems0)
            return carry

        nfull = jnp.where(s == NS - 1, RLAST // K, RSPAN // K)
        lax.fori_loop(0, nfull, fill_acc, 0)

        @pl.when(s < NS - 1)
        def _():
            pltpu.async_copy(rows0.at[pl.ds(0, RSPAN - (RSPAN // K) * K)],
                             acc_sh.at[pl.ds(roff + (RSPAN // K) * K,
                                             RSPAN - (RSPAN // K) * K)],
                             sems0)

        def drain_fill(i, carry):
            pltpu.make_async_copy(
                rows0, acc_sh.at[pl.ds(roff, K)], sems0).wait()
            return carry

        lax.fori_loop(0, nfull, drain_fill, 0)

        @pl.when(s < NS - 1)
        def _():
            pltpu.make_async_copy(
                rows0.at[pl.ds(0, RSPAN - (RSPAN // K) * K)],
                acc_sh.at[pl.ds(roff, RSPAN - (RSPAN // K) * K)],
                sems0).wait()

        plsc.subcore_barrier()

        def start_idx(g, idx, hbm, semi):
            pltpu.async_copy(hbm.at[pl.ds(ebase + g * K, K)], idx, semi)

        def wait_idx(idx, hbm, semi):
            pltpu.make_async_copy(hbm.at[pl.ds(ebase, K)], idx, semi).wait()

        def start_attr(g, attr_b, sema):
            pltpu.async_copy(attr_hbm.at[pl.ds((ebase + g * K) * 16, K * 16)],
                             attr_b, sema)

        def wait_attr(attr_b, sema):
            pltpu.make_async_copy(attr_hbm.at[pl.ds(ebase * 16, K * 16)],
                                  attr_b, sema).wait()

        def start_gather(srcidx, rows, semg):
            pltpu.async_copy(feat_hbm.at[srcidx], rows, semg)

        def wait_gather(srcidx, rows, semg):
            pltpu.make_async_copy(feat_hbm.at[srcidx], rows, semg).wait()

        def scale(rows, attr_b):
            def row_body(r, carry):
                bc = attr_b[pl.ds(r * 16, 16)]
                for j in range(D // 16):
                    sl = pl.ds(j * 16, 16)
                    rows[r, sl] = rows[r, sl] * bc
                return carry

            lax.fori_loop(0, K, row_body, 0)

        def start_scatter(dstidx, rows, sems):
            pltpu.async_copy(rows, acc_sh.at[dstidx], sems, add=True)

        def wait_scatter(dstidx, rows, sems):
            pltpu.make_async_copy(rows, acc_sh.at[dstidx], sems).wait()

        start_idx(0, srcidx0, src_hbm, semi0)
        start_idx(1, srcidx1, src_hbm, semi1)
        start_idx(0, dstidx0, dst_hbm, semd0)
        start_idx(1, dstidx1, dst_hbm, semd1)
        start_attr(0, attr0, sema0)
        start_attr(1, attr1, sema1)
        wait_idx(srcidx0, src_hbm, semi0)
        start_gather(srcidx0, rows0, semg0)
        wait_idx(srcidx1, src_hbm, semi1)
        start_gather(srcidx1, rows1, semg1)

        def pair_body(m, carry):
            e0 = 2 * m
            e1 = e0 + 1
            not_last = m < PAIRS - 1

            wait_gather(srcidx0, rows0, semg0)

            @pl.when(not_last)
            def _():
                start_idx(e0 + 2, srcidx0, src_hbm, semi0)

            wait_attr(attr0, sema0)
            scale(rows0, attr0)

            @pl.when(not_last)
            def _():
                start_attr(e0 + 2, attr0, sema0)

            wait_idx(dstidx0, dst_hbm, semd0)
            start_scatter(dstidx0, rows0, sems0)

            wait_gather(srcidx1, rows1, semg1)

            @pl.when(not_last)
            def _():
                start_idx(e1 + 2, srcidx1, src_hbm, semi1)

            wait_attr(attr1, sema1)
            scale(rows1, attr1)

            @pl.when(not_last)
            def _():
                start_attr(e1 + 2, attr1, sema1)

            wait_idx(dstidx1, dst_hbm, semd1)
            start_scatter(dstidx1, rows1, sems1)

            @pl.when(not_last)
            def _():
                wait_scatter(dstidx0, rows0, sems0)
                start_idx(e0 + 2, dstidx0, dst_hbm, semd0)
                wait_idx(srcidx0, src_hbm, semi0)
                start_gather(srcidx0, rows0, semg0)
                wait_scatter(dstidx1, rows1, sems1)
                start_idx(e1 + 2, dstidx1, dst_hbm, semd1)
                wait_idx(srcidx1, src_hbm, semi1)
                start_gather(srcidx1, rows1, semg1)

            return carry

        lax.fori_loop(0, PAIRS, pair_body, 0)
        wait_scatter(dstidx0, rows0, sems0)
        wait_scatter(dstidx1, rows1, sems1)
        plsc.subcore_barrier()

        obase = c * N + roff

        @pl.when(s < NS - 1)
        def _():
            pltpu.sync_copy(acc_sh.at[pl.ds(roff, RSPAN)],
                            out_hbm.at[pl.ds(obase, RSPAN)])

        @pl.when(s == NS - 1)
        def _():
            pltpu.sync_copy(acc_sh.at[pl.ds(roff, RLAST)],
                            out_hbm.at[pl.ds(obase, RLAST)])

    return k(feat, esrc, edst3, eattr3)


def _tc_post(parts, self_out, tp_w_row, w2):
    bm = 1000
    cos_a = math.cos(MIXING_ANGLE)
    sin_scaled = math.sin(MIXING_ANGLE) / math.sqrt(NUM_NEIGHBORS * D)

    def body(p0_ref, p1_ref, self_ref, tpw_ref, w2_ref, o_ref):
        sacc = p0_ref[...] + p1_ref[...]
        nf = sacc * tpw_ref[...]
        conv = jnp.dot(nf, w2_ref[...], preferred_element_type=jnp.float32)
        o_ref[...] = cos_a * self_ref[...] + sin_scaled * conv

    nb = N // bm
    return pl.pallas_call(
        body,
        grid=(nb,),
        in_specs=[
            pl.BlockSpec((bm, D), lambda i: (i, 0)),
            pl.BlockSpec((bm, D), lambda i: (i + nb, 0)),
            pl.BlockSpec((bm, D), lambda i: (i, 0)),
            pl.BlockSpec((1, D), lambda i: (0, 0)),
            pl.BlockSpec((D, D), lambda i: (0, 0)),
        ],
        out_specs=pl.BlockSpec((bm, D), lambda i: (i, 0)),
        out_shape=jax.ShapeDtypeStruct((N, D), jnp.float32),
    )(parts, parts, self_out, tp_w_row, w2)


def kernel(node_input, edge_src, edge_dst, edge_attr, W1, tp_w, W2):
    feat, self_out = _tc_pre(node_input, W1)
    attr_flat = jnp.repeat(edge_attr[:, 0], 16)
    parts = _sc_gather_scatter(feat, edge_src, edge_dst, attr_flat)
    return _tc_post(parts, self_out, tp_w.reshape(1, D), W2)


# speedup vs baseline: 1.6539x; 1.6539x over previous
import functools
import math

import jax
import jax.numpy as jnp
from jax import lax
from jax.experimental import pallas as pl
from jax.experimental.pallas import tpu as pltpu
from jax.experimental.pallas import tpu_sc as plsc

N = 10000
D = 128
E = 320000
NUM_NEIGHBORS = 32.0
MIXING_ANGLE = math.pi / 8.0

NC = 2
NS = 16
NW = NC * NS
EPW = E // NW
K = 40
CHUNKS = EPW // K
PAIRS = CHUNKS // 2
RSPAN = 624
RLAST = N - 15 * RSPAN

def _tc_pre(x, w1):
    bm = 1000

    def body(x_ref, w1_ref, feat_ref, self_ref):
        t = jnp.dot(x_ref[...], w1_ref[...], preferred_element_type=jnp.float32)
        t = t * (1.0 / math.sqrt(D))
        feat_ref[...] = t[:, :D]
        self_ref[...] = t[:, D:]

    return pl.pallas_call(
        body,
        grid=(N // bm,),
        in_specs=[
            pl.BlockSpec((bm, D), lambda i: (i, 0)),
            pl.BlockSpec((D, 2 * D), lambda i: (0, 0)),
        ],
        out_specs=[
            pl.BlockSpec((bm, D), lambda i: (i, 0)),
            pl.BlockSpec((bm, D), lambda i: (i, 0)),
        ],
        out_shape=[
            jax.ShapeDtypeStruct((N, D), jnp.float32),
            jax.ShapeDtypeStruct((N, D), jnp.float32),
        ],
    )(x, w1)


def _sc_gather_scatter(feat, esrc, edst, eattr):
    mesh = plsc.VectorSubcoreMesh(core_axis_name="c", subcore_axis_name="s")

    @functools.partial(
        pl.kernel,
        mesh=mesh,
        out_type=jax.ShapeDtypeStruct((NC * N, D), jnp.float32),
        scratch_types=[
            pltpu.VMEM((K,), jnp.int32),
            pltpu.VMEM((K,), jnp.int32),
            pltpu.VMEM((K,), jnp.int32),
            pltpu.VMEM((K,), jnp.int32),
            pltpu.VMEM((K, 16), jnp.float32),
            pltpu.VMEM((K, 16), jnp.float32),
            pltpu.VMEM((K, D), jnp.float32),
            pltpu.VMEM((K, D), jnp.float32),
            pltpu.VMEM((K, D), jnp.float32),
            pltpu.VMEM((K, D), jnp.float32),
            pltpu.VMEM_SHARED((N, D), jnp.float32),
            pltpu.SemaphoreType.DMA,
            pltpu.SemaphoreType.DMA,
            pltpu.SemaphoreType.DMA,
            pltpu.SemaphoreType.DMA,
            pltpu.SemaphoreType.DMA,
            pltpu.SemaphoreType.DMA,
            pltpu.SemaphoreType.DMA,
            pltpu.SemaphoreType.DMA,
            pltpu.SemaphoreType.DMA,
            pltpu.SemaphoreType.DMA,
        ],
    )
    def k(feat_hbm, src_hbm, dst_hbm, attr_hbm, out_hbm,
          srcidx0, srcidx1, dstidx0, dstidx1, attr0, attr1,
          rowsb0, rowsb1, rowsf0, rowsf1, acc_sh,
          semi0, semi1, semd0, semd1, sema0, sema1,
          semg0, semg1, sems0, sems1):
        c = lax.axis_index("c")
        s = lax.axis_index("s")
        tid = c * NS + s
        ebase = tid * EPW

        roff = s * RSPAN

        def zero_rows(r, carry):
            for j in range(D // 16):
                rowsf0[r, pl.ds(j * 16, 16)] = jnp.zeros((16,), jnp.float32)
            return carry

        lax.fori_loop(0, K, zero_rows, 0)

        def fill_acc(i, carry):
            pltpu.async_copy(rowsf0, acc_sh.at[pl.ds(roff + i * K, K)], sems0)
            return carry

        nfull = jnp.where(s == NS - 1, RLAST // K, RSPAN // K)
        lax.fori_loop(0, nfull, fill_acc, 0)

        @pl.when(s < NS - 1)
        def _():
            pltpu.async_copy(rowsf0.at[pl.ds(0, RSPAN - (RSPAN // K) * K)],
                             acc_sh.at[pl.ds(roff + (RSPAN // K) * K,
                                             RSPAN - (RSPAN // K) * K)],
                             sems0)

        def drain_fill(i, carry):
            pltpu.make_async_copy(
                rowsf0, acc_sh.at[pl.ds(roff, K)], sems0).wait()
            return carry

        lax.fori_loop(0, nfull, drain_fill, 0)

        @pl.when(s < NS - 1)
        def _():
            pltpu.make_async_copy(
                rowsf0.at[pl.ds(0, RSPAN - (RSPAN // K) * K)],
                acc_sh.at[pl.ds(roff, RSPAN - (RSPAN // K) * K)],
                sems0).wait()

        plsc.subcore_barrier()

        def start_idx(g, idx, hbm, semi):
            pltpu.async_copy(hbm.at[pl.ds(ebase + g * K, K)], idx, semi)

        def wait_idx(idx, hbm, semi):
            pltpu.make_async_copy(hbm.at[pl.ds(ebase, K)], idx, semi).wait()

        def start_attr(g, attr_b, sema):
            pltpu.async_copy(attr_hbm.at[tid, g], attr_b, sema)

        def wait_attr(attr_b, sema):
            pltpu.make_async_copy(attr_hbm.at[tid, 0], attr_b, sema).wait()

        def start_gather(srcidx, rowsb, semg):
            pltpu.async_copy(feat_hbm.at[srcidx], rowsb, semg)

        def wait_gather(srcidx, rowsb, semg):
            pltpu.make_async_copy(feat_hbm.at[srcidx], rowsb, semg).wait()

        def scale(rowsb, rowsf, attr_b):
            def row_body(r, carry):
                bc = attr_b[r]
                for j in range(D // 16):
                    sl = pl.ds(j * 16, 16)
                    rowsf[r, sl] = rowsb[r, sl] * bc
                return carry

            lax.fori_loop(0, K, row_body, 0)

        def start_scatter(dstidx, rowsf, sems):
            pltpu.async_copy(rowsf, acc_sh.at[dstidx], sems, add=True)

        def wait_scatter(dstidx, rowsf, sems):
            pltpu.make_async_copy(rowsf, acc_sh.at[dstidx], sems).wait()

        start_idx(0, srcidx0, src_hbm, semi0)
        start_idx(1, srcidx1, src_hbm, semi1)
        start_attr(0, attr0, sema0)
        start_attr(1, attr1, sema1)
        wait_idx(srcidx0, src_hbm, semi0)
        start_gather(srcidx0, rowsb0, semg0)
        wait_idx(srcidx1, src_hbm, semi1)
        start_gather(srcidx1, rowsb1, semg1)

        def half_iter(m, e0, srcidx, dstidx, attr_b, rowsb, rowsf,
                      semi, semd, sema, semg, sems):
            not_last = m < PAIRS - 1

            wait_gather(srcidx, rowsb, semg)

            @pl.when(not_last)
            def _():
                start_idx(e0 + 2, srcidx, src_hbm, semi)

            @pl.when(m > 0)
            def _():
                wait_scatter(dstidx, rowsf, sems)

            start_idx(e0, dstidx, dst_hbm, semd)
            wait_attr(attr_b, sema)
            scale(rowsb, rowsf, attr_b)

            @pl.when(not_last)
            def _():
                start_attr(e0 + 2, attr_b, sema)
                wait_idx(srcidx, src_hbm, semi)
                start_gather(srcidx, rowsb, semg)

            wait_idx(dstidx, dst_hbm, semd)
            start_scatter(dstidx, rowsf, sems)

        def pair_body(m, carry):
            half_iter(m, 2 * m, srcidx0, dstidx0, attr0, rowsb0, rowsf0,
                      semi0, semd0, sema0, semg0, sems0)
            half_iter(m, 2 * m + 1, srcidx1, dstidx1, attr1, rowsb1, rowsf1,
                      semi1, semd1, sema1, semg1, sems1)
            return carry

        lax.fori_loop(0, PAIRS, pair_body, 0)
        wait_scatter(dstidx0, rowsf0, sems0)
        wait_scatter(dstidx1, rowsf1, sems1)
        plsc.subcore_barrier()

        obase = c * N + roff

        @pl.when(s < NS - 1)
        def _():
            pltpu.sync_copy(acc_sh.at[pl.ds(roff, RSPAN)],
                            out_hbm.at[pl.ds(obase, RSPAN)])

        @pl.when(s == NS - 1)
        def _():
            pltpu.sync_copy(acc_sh.at[pl.ds(roff, RLAST)],
                            out_hbm.at[pl.ds(obase, RLAST)])

    return k(feat, esrc, edst, eattr)


def _tc_post(parts, self_out, tp_w_row, w2):
    bm = 1000
    cos_a = math.cos(MIXING_ANGLE)
    sin_scaled = math.sin(MIXING_ANGLE) / math.sqrt(NUM_NEIGHBORS * D)

    def body(p0_ref, p1_ref, self_ref, tpw_ref, w2_ref, o_ref):
        sacc = p0_ref[...] + p1_ref[...]
        nf = sacc * tpw_ref[...]
        conv = jnp.dot(nf, w2_ref[...], preferred_element_type=jnp.float32)
        o_ref[...] = cos_a * self_ref[...] + sin_scaled * conv

    nb = N // bm
    return pl.pallas_call(
        body,
        grid=(nb,),
        in_specs=[
            pl.BlockSpec((bm, D), lambda i: (i, 0)),
            pl.BlockSpec((bm, D), lambda i: (i + nb, 0)),
            pl.BlockSpec((bm, D), lambda i: (i, 0)),
            pl.BlockSpec((1, D), lambda i: (0, 0)),
            pl.BlockSpec((D, D), lambda i: (0, 0)),
        ],
        out_specs=pl.BlockSpec((bm, D), lambda i: (i, 0)),
        out_shape=jax.ShapeDtypeStruct((N, D), jnp.float32),
    )(parts, parts, self_out, tp_w_row, w2)


def kernel(node_input, edge_src, edge_dst, edge_attr, W1, tp_w, W2):
    feat, self_out = _tc_pre(node_input, W1)
    eattr4 = jnp.broadcast_to(edge_attr, (E, 16)).reshape(NW, CHUNKS, K, 16)
    parts = _sc_gather_scatter(feat, edge_src, edge_dst, eattr4)
    return _tc_post(parts, self_out, tp_w.reshape(1, D), W2)
